# Initial kernel scaffold; baseline (speedup 1.0000x reference)
import functools

import jax
import jax.numpy as jnp
from jax import lax
from jax.experimental import pallas as pl
from jax.experimental.pallas import tpu as pltpu
from jax.experimental.pallas import tpu_sc as plsc

N = 10000
E = 320000
NCORES = 2
NSUB = 16
NTILES = NCORES * NSUB
EDGES_PER_TILE = E // NTILES
CHUNK = 80
NCHUNKS = EDGES_PER_TILE // CHUNK
ROWS_PER_TILE = N // NSUB
ZROWS = 125


def _make_agg(D):
    mesh = plsc.VectorSubcoreMesh(core_axis_name="c", subcore_axis_name="s")

    @functools.partial(
        pl.kernel,
        out_type=jax.ShapeDtypeStruct((NCORES, N, D), jnp.float32),
        mesh=mesh,
        scratch_types=[
            pltpu.VMEM((CHUNK,), jnp.int32),
            pltpu.VMEM((CHUNK,), jnp.int32),
            pltpu.VMEM((CHUNK,), jnp.float32),
            pltpu.VMEM((CHUNK, D), jnp.float32),
            pltpu.VMEM((ZROWS, D), jnp.float32),
            pltpu.VMEM_SHARED((N, D), jnp.float32),
            pltpu.SemaphoreType.DMA,
        ],
    )
    def agg(sup_hbm, src_hbm, dst_hbm, w_hbm, out_hbm,
            sidx, didx, wbuf, rows, zbuf, acc, sem):
        c = lax.axis_index("c")
        s = lax.axis_index("s")

        zv = jnp.zeros((16,), jnp.float32)

        def zrow(r, carry):
            for cb in range(D // 16):
                zbuf[r, pl.ds(cb * 16, 16)] = zv
            return carry

        lax.fori_loop(0, ZROWS, zrow, 0)

        row0 = s * ROWS_PER_TILE
        for z in range(ROWS_PER_TILE // ZROWS):
            pltpu.sync_copy(zbuf, acc.at[pl.ds(row0 + z * ZROWS, ZROWS)])
        plsc.subcore_barrier()

        base = (c * NSUB + s) * EDGES_PER_TILE

        def chunk_body(j, carry):
            off = base + j * CHUNK
            pltpu.sync_copy(src_hbm.at[pl.ds(off, CHUNK)], sidx)
            pltpu.sync_copy(dst_hbm.at[pl.ds(off, CHUNK)], didx)
            pltpu.sync_copy(w_hbm.at[pl.ds(off, CHUNK)], wbuf)
            pltpu.async_copy(sup_hbm.at[sidx], rows, sem).wait()

            def mrow(r, rcarry):
                w = wbuf[r]
                for cb in range(D // 16):
                    rows[r, pl.ds(cb * 16, 16)] = rows[r, pl.ds(cb * 16, 16)] * w
                return rcarry

            lax.fori_loop(0, CHUNK, mrow, 0)
            pltpu.sync_copy(rows, acc.at[didx], add=True)
            return carry

        lax.fori_loop(0, NCHUNKS, chunk_body, 0)
        plsc.subcore_barrier()
        pltpu.sync_copy(acc.at[pl.ds(row0, ROWS_PER_TILE)],
                        out_hbm.at[c, pl.ds(row0, ROWS_PER_TILE)])

    return agg


_agg64 = _make_agg(64)
_agg32 = _make_agg(32)
_agg16 = _make_agg(16)

_ROWS_BLK = 1000
_GRID = N // _ROWS_BLK


def _mm_first(x, W):
    K, Dout = W.shape

    def body(x_ref, w_ref, o_ref):
        o_ref[...] = lax.dot_general(
            x_ref[...], w_ref[...], (((1,), (0,)), ((), ())),
            precision=lax.Precision.HIGHEST)

    return pl.pallas_call(
        body,
        grid=(_GRID,),
        in_specs=[pl.BlockSpec((_ROWS_BLK, K), lambda i: (i, 0)),
                  pl.BlockSpec((K, Dout), lambda i: (0, 0))],
        out_specs=pl.BlockSpec((_ROWS_BLK, Dout), lambda i: (i, 0)),
        out_shape=jax.ShapeDtypeStruct((N, Dout), jnp.float32),
    )(x, W)


def _mm_fused(parts, b, W):
    Din, Dout = W.shape

    def body(p_ref, b_ref, w_ref, o_ref):
        h = jnp.maximum(p_ref[0] + p_ref[1] + b_ref[...], 0.0)
        o_ref[...] = lax.dot_general(
            h, w_ref[...], (((1,), (0,)), ((), ())),
            precision=lax.Precision.HIGHEST)

    return pl.pallas_call(
        body,
        grid=(_GRID,),
        in_specs=[pl.BlockSpec((NCORES, _ROWS_BLK, Din), lambda i: (0, i, 0)),
                  pl.BlockSpec((1, Din), lambda i: (0, 0)),
                  pl.BlockSpec((Din, Dout), lambda i: (0, 0))],
        out_specs=pl.BlockSpec((_ROWS_BLK, Dout), lambda i: (i, 0)),
        out_shape=jax.ShapeDtypeStruct((N, Dout), jnp.float32),
    )(parts, b, W)


def _final_add(parts, b):
    Din = parts.shape[-1]

    def body(p_ref, b_ref, o_ref):
        o_ref[...] = p_ref[0] + p_ref[1] + b_ref[...]

    return pl.pallas_call(
        body,
        grid=(_GRID,),
        in_specs=[pl.BlockSpec((NCORES, _ROWS_BLK, Din), lambda i: (0, i, 0)),
                  pl.BlockSpec((1, Din), lambda i: (0, 0))],
        out_specs=pl.BlockSpec((_ROWS_BLK, Din), lambda i: (i, 0)),
        out_shape=jax.ShapeDtypeStruct((N, Din), jnp.float32),
    )(parts, b)


def kernel(x, edge_index, edge_weight, W1, b1, W2, b2, W3, b3):
    src = edge_index[0]
    dst = edge_index[1]
    W3p = jnp.pad(W3, ((0, 0), (0, 16 - W3.shape[1])))
    b1r = b1.reshape(1, -1)
    b2r = b2.reshape(1, -1)
    b3p = jnp.pad(b3, (0, 16 - b3.shape[0])).reshape(1, -1)

    s1 = _mm_first(x, W1)
    p1 = _agg64(s1, src, dst, edge_weight)
    s2 = _mm_fused(p1, b1r, W2)
    p2 = _agg32(s2, src, dst, edge_weight)
    s3 = _mm_fused(p2, b2r, W3p)
    p3 = _agg16(s3, src, dst, edge_weight)
    outp = _final_add(p3, b3p)
    return outp[:, :W3.shape[1]]


# speedup vs baseline: 4.4498x; 4.4498x over previous
import functools

import jax
import jax.numpy as jnp
from jax import lax
from jax.experimental import pallas as pl
from jax.experimental.pallas import tpu as pltpu
from jax.experimental.pallas import tpu_sc as plsc

N = 10000
E = 320000
NCORES = 2
NSUB = 16
NTILES = NCORES * NSUB
EDGES_PER_TILE = E // NTILES
CHUNK = 80
NCHUNKS = EDGES_PER_TILE // CHUNK
SLAB = 624
TAIL = N - NSUB * SLAB
ZROWS = 208


def _make_agg(D):
    mesh = plsc.VectorSubcoreMesh(core_axis_name="c", subcore_axis_name="s")

    @functools.partial(
        pl.kernel,
        out_type=jax.ShapeDtypeStruct((NCORES, N, D), jnp.float32),
        mesh=mesh,
        compiler_params=pltpu.CompilerParams(use_tc_tiling_on_sc=False),
        scratch_types=[
            pltpu.VMEM((CHUNK,), jnp.int32),
            pltpu.VMEM((CHUNK,), jnp.int32),
            pltpu.VMEM((CHUNK,), jnp.float32),
            pltpu.VMEM((CHUNK, D), jnp.float32),
            pltpu.VMEM((ZROWS, D), jnp.float32),
            pltpu.VMEM_SHARED((N, D), jnp.float32),
            pltpu.SemaphoreType.DMA,
        ],
    )
    def agg(sup_hbm, src_hbm, dst_hbm, w_hbm, out_hbm,
            sidx, didx, wbuf, rows, zbuf, acc, sem):
        c = lax.axis_index("c")
        s = lax.axis_index("s")

        zv = jnp.zeros((16,), jnp.float32)

        def zrow(r, carry):
            for cb in range(D // 16):
                zbuf[r, pl.ds(cb * 16, 16)] = zv
            return carry

        lax.fori_loop(0, ZROWS, zrow, 0)

        row0 = s * SLAB
        for z in range(SLAB // ZROWS):
            pltpu.sync_copy(zbuf, acc.at[pl.ds(row0 + z * ZROWS, ZROWS)])

        @pl.when(s == NSUB - 1)
        def _zero_tail():
            pltpu.sync_copy(zbuf.at[pl.ds(0, TAIL)],
                            acc.at[pl.ds(NSUB * SLAB, TAIL)])

        plsc.subcore_barrier()

        base = (c * NSUB + s) * EDGES_PER_TILE

        def chunk_body(j, carry):
            off = base + j * CHUNK
            pltpu.sync_copy(src_hbm.at[pl.ds(off, CHUNK)], sidx)
            pltpu.sync_copy(dst_hbm.at[pl.ds(off, CHUNK)], didx)
            pltpu.sync_copy(w_hbm.at[pl.ds(off, CHUNK)], wbuf)
            pltpu.async_copy(sup_hbm.at[sidx], rows, sem).wait()

            def mgrp(g, rcarry):
                wv = wbuf[pl.ds(g * 16, 16)]
                for i in range(16):
                    w = wv[i]
                    r = g * 16 + i
                    for cb in range(D // 16):
                        rows[r, pl.ds(cb * 16, 16)] = (
                            rows[r, pl.ds(cb * 16, 16)] * w)
                return rcarry

            lax.fori_loop(0, CHUNK // 16, mgrp, 0)
            pltpu.sync_copy(rows, acc.at[didx], add=True)
            return carry

        lax.fori_loop(0, NCHUNKS, chunk_body, 0)
        plsc.subcore_barrier()
        pltpu.sync_copy(acc.at[pl.ds(row0, SLAB)],
                        out_hbm.at[c, pl.ds(row0, SLAB)])

        @pl.when(s == NSUB - 1)
        def _drain_tail():
            pltpu.sync_copy(acc.at[pl.ds(NSUB * SLAB, TAIL)],
                            out_hbm.at[c, pl.ds(NSUB * SLAB, TAIL)])

    return agg


_agg64 = _make_agg(64)
_agg32 = _make_agg(32)
_agg16 = _make_agg(16)

_ROWS_BLK = 1000
_GRID = N // _ROWS_BLK


def _mm_first(x, W):
    K, Dout = W.shape

    def body(x_ref, w_ref, o_ref):
        o_ref[...] = lax.dot_general(
            x_ref[...], w_ref[...], (((1,), (0,)), ((), ())),
            precision=lax.Precision.HIGHEST)

    return pl.pallas_call(
        body,
        grid=(_GRID,),
        in_specs=[pl.BlockSpec((_ROWS_BLK, K), lambda i: (i, 0)),
                  pl.BlockSpec((K, Dout), lambda i: (0, 0))],
        out_specs=pl.BlockSpec((_ROWS_BLK, Dout), lambda i: (i, 0)),
        out_shape=jax.ShapeDtypeStruct((N, Dout), jnp.float32),
    )(x, W)


def _mm_fused(parts, b, W):
    Din, Dout = W.shape

    def body(p_ref, b_ref, w_ref, o_ref):
        h = jnp.maximum(p_ref[0] + p_ref[1] + b_ref[...], 0.0)
        o_ref[...] = lax.dot_general(
            h, w_ref[...], (((1,), (0,)), ((), ())),
            precision=lax.Precision.HIGHEST)

    return pl.pallas_call(
        body,
        grid=(_GRID,),
        in_specs=[pl.BlockSpec((NCORES, _ROWS_BLK, Din), lambda i: (0, i, 0)),
                  pl.BlockSpec((1, Din), lambda i: (0, 0)),
                  pl.BlockSpec((Din, Dout), lambda i: (0, 0))],
        out_specs=pl.BlockSpec((_ROWS_BLK, Dout), lambda i: (i, 0)),
        out_shape=jax.ShapeDtypeStruct((N, Dout), jnp.float32),
    )(parts, b, W)


def _final_add(parts, b):
    Din = parts.shape[-1]

    def body(p_ref, b_ref, o_ref):
        o_ref[...] = p_ref[0] + p_ref[1] + b_ref[...]

    return pl.pallas_call(
        body,
        grid=(_GRID,),
        in_specs=[pl.BlockSpec((NCORES, _ROWS_BLK, Din), lambda i: (0, i, 0)),
                  pl.BlockSpec((1, Din), lambda i: (0, 0))],
        out_specs=pl.BlockSpec((_ROWS_BLK, Din), lambda i: (i, 0)),
        out_shape=jax.ShapeDtypeStruct((N, Din), jnp.float32),
    )(parts, b)


def kernel(x, edge_index, edge_weight, W1, b1, W2, b2, W3, b3):
    src = edge_index[0]
    dst = edge_index[1]
    W3p = jnp.pad(W3, ((0, 0), (0, 16 - W3.shape[1])))
    b1r = b1.reshape(1, -1)
    b2r = b2.reshape(1, -1)
    b3p = jnp.pad(b3, (0, 16 - b3.shape[0])).reshape(1, -1)

    s1 = _mm_first(x, W1)
    p1 = _agg64(s1, src, dst, edge_weight)
    s2 = _mm_fused(p1, b1r, W2)
    p2 = _agg32(s2, src, dst, edge_weight)
    s3 = _mm_fused(p2, b2r, W3p)
    p3 = _agg16(s3, src, dst, edge_weight)
    outp = _final_add(p3, b3p)
    return outp[:, :W3.shape[1]]


# speedup vs baseline: 8.2716x; 1.8588x over previous
import functools

import jax
import jax.numpy as jnp
from jax import lax
from jax.experimental import pallas as pl
from jax.experimental.pallas import tpu as pltpu
from jax.experimental.pallas import tpu_sc as plsc

N = 10000
E = 320000
NCORES = 2
NSUB = 16
NTILES = NCORES * NSUB
EDGES_PER_TILE = E // NTILES
CHUNK = 80
NCHUNKS = EDGES_PER_TILE // CHUNK
SLAB = 624
TAIL = N - NSUB * SLAB
ZROWS = 208


NPAIRS = NCHUNKS // 2


def _make_agg(D):
    mesh = plsc.VectorSubcoreMesh(core_axis_name="c", subcore_axis_name="s")

    @functools.partial(
        pl.kernel,
        out_type=jax.ShapeDtypeStruct((NCORES, N, D), jnp.float32),
        mesh=mesh,
        compiler_params=pltpu.CompilerParams(use_tc_tiling_on_sc=False,
                                             needs_layout_passes=False),
        scratch_types=[
            pltpu.VMEM((2, 3, CHUNK), jnp.int32),
            pltpu.VMEM((2, CHUNK), jnp.int32),
            pltpu.VMEM((2, CHUNK, D), jnp.float32),
            pltpu.VMEM((ZROWS, D), jnp.float32),
            pltpu.VMEM_SHARED((N, D), jnp.float32),
            pltpu.SemaphoreType.DMA,
            pltpu.SemaphoreType.DMA,
            pltpu.SemaphoreType.DMA,
            pltpu.SemaphoreType.DMA,
            pltpu.SemaphoreType.DMA,
            pltpu.SemaphoreType.DMA,
        ],
    )
    def agg(sup_hbm, ed_hbm, out_hbm,
            ebuf, dbuf, rows, zbuf, acc,
            isem0, isem1, gsem0, gsem1, ssem0, ssem1):
        c = lax.axis_index("c")
        s = lax.axis_index("s")
        isem = (isem0, isem1)
        gsem = (gsem0, gsem1)
        ssem = (ssem0, ssem1)

        zv = jnp.zeros((16,), jnp.float32)

        def zrow(r, carry):
            for cb in range(D // 16):
                zbuf[r, pl.ds(cb * 16, 16)] = zv
            return carry

        lax.fori_loop(0, ZROWS, zrow, 0)

        row0 = s * SLAB
        for z in range(SLAB // ZROWS):
            pltpu.sync_copy(zbuf, acc.at[pl.ds(row0 + z * ZROWS, ZROWS)])

        @pl.when(s == NSUB - 1)
        def _zero_tail():
            pltpu.sync_copy(zbuf.at[pl.ds(0, TAIL)],
                            acc.at[pl.ds(NSUB * SLAB, TAIL)])

        plsc.subcore_barrier()

        base = (c * NSUB + s) * EDGES_PER_TILE

        def idx_cp(j, b):
            off = base + j * CHUNK
            return pltpu.make_async_copy(
                ed_hbm.at[:, pl.ds(off, CHUNK)], ebuf.at[b], isem[b])

        def gat_cp(b):
            return pltpu.make_async_copy(
                sup_hbm.at[ebuf.at[b, 0]], rows.at[b], gsem[b])

        def sct_start(b):
            pltpu.async_copy(rows.at[b], acc.at[dbuf.at[b]], ssem[b],
                             add=True)

        def sct_wait(b):
            pltpu.make_async_copy(rows.at[b], acc.at[dbuf.at[b]],
                                  ssem[b]).wait()

        def process(b):
            def grp(g, carry):
                dbuf[b, pl.ds(g * 16, 16)] = ebuf[b, 1, pl.ds(g * 16, 16)]
                wv = plsc.bitcast(ebuf[b, 2, pl.ds(g * 16, 16)], jnp.float32)
                for i in range(16):
                    w = wv[i]
                    r = g * 16 + i
                    for cb in range(D // 16):
                        rows[b, r, pl.ds(cb * 16, 16)] = (
                            rows[b, r, pl.ds(cb * 16, 16)] * w)
                return carry

            lax.fori_loop(0, CHUNK // 16, grp, 0)

        idx_cp(0, 0).start()
        idx_cp(0, 0).wait()
        gat_cp(0).start()
        idx_cp(1, 1).start()

        def pair(p, carry):
            gat_cp(0).wait()
            process(0)
            sct_start(0)
            idx_cp(2 * p + 2, 0).start()
            idx_cp(2 * p + 1, 1).wait()

            @pl.when(p > 0)
            def _():
                sct_wait(1)

            gat_cp(1).start()

            gat_cp(1).wait()
            process(1)
            sct_start(1)

            @pl.when(p < NPAIRS - 1)
            def _():
                idx_cp(2 * p + 3, 1).start()

            idx_cp(2 * p + 2, 0).wait()
            sct_wait(0)
            gat_cp(0).start()
            return carry

        lax.fori_loop(0, NPAIRS, pair, 0)

        gat_cp(0).wait()
        process(0)
        sct_start(0)
        sct_wait(0)
        sct_wait(1)

        plsc.subcore_barrier()
        pltpu.sync_copy(acc.at[pl.ds(row0, SLAB)],
                        out_hbm.at[c, pl.ds(row0, SLAB)])

        @pl.when(s == NSUB - 1)
        def _drain_tail():
            pltpu.sync_copy(acc.at[pl.ds(NSUB * SLAB, TAIL)],
                            out_hbm.at[c, pl.ds(NSUB * SLAB, TAIL)])

    return agg


_agg64 = _make_agg(64)
_agg32 = _make_agg(32)
_agg16 = _make_agg(16)

_ROWS_BLK = 1000
_GRID = N // _ROWS_BLK


def _mm_first(x, W):
    K, Dout = W.shape

    def body(x_ref, w_ref, o_ref):
        o_ref[...] = lax.dot_general(
            x_ref[...], w_ref[...], (((1,), (0,)), ((), ())),
            precision=lax.Precision.HIGHEST)

    return pl.pallas_call(
        body,
        grid=(_GRID,),
        in_specs=[pl.BlockSpec((_ROWS_BLK, K), lambda i: (i, 0)),
                  pl.BlockSpec((K, Dout), lambda i: (0, 0))],
        out_specs=pl.BlockSpec((_ROWS_BLK, Dout), lambda i: (i, 0)),
        out_shape=jax.ShapeDtypeStruct((N, Dout), jnp.float32),
    )(x, W)


def _mm_fused(parts, b, W):
    Din, Dout = W.shape

    def body(p_ref, b_ref, w_ref, o_ref):
        h = jnp.maximum(p_ref[0] + p_ref[1] + b_ref[...], 0.0)
        o_ref[...] = lax.dot_general(
            h, w_ref[...], (((1,), (0,)), ((), ())),
            precision=lax.Precision.HIGHEST)

    return pl.pallas_call(
        body,
        grid=(_GRID,),
        in_specs=[pl.BlockSpec((NCORES, _ROWS_BLK, Din), lambda i: (0, i, 0)),
                  pl.BlockSpec((1, Din), lambda i: (0, 0)),
                  pl.BlockSpec((Din, Dout), lambda i: (0, 0))],
        out_specs=pl.BlockSpec((_ROWS_BLK, Dout), lambda i: (i, 0)),
        out_shape=jax.ShapeDtypeStruct((N, Dout), jnp.float32),
    )(parts, b, W)


def _final_add(parts, b):
    Din = parts.shape[-1]

    def body(p_ref, b_ref, o_ref):
        o_ref[...] = p_ref[0] + p_ref[1] + b_ref[...]

    return pl.pallas_call(
        body,
        grid=(_GRID,),
        in_specs=[pl.BlockSpec((NCORES, _ROWS_BLK, Din), lambda i: (0, i, 0)),
                  pl.BlockSpec((1, Din), lambda i: (0, 0))],
        out_specs=pl.BlockSpec((_ROWS_BLK, Din), lambda i: (i, 0)),
        out_shape=jax.ShapeDtypeStruct((N, Din), jnp.float32),
    )(parts, b)


def kernel(x, edge_index, edge_weight, W1, b1, W2, b2, W3, b3):
    wbits = jax.lax.bitcast_convert_type(edge_weight, jnp.int32)
    ed = jnp.concatenate([edge_index, wbits[None]], axis=0)
    W3p = jnp.pad(W3, ((0, 0), (0, 16 - W3.shape[1])))
    b1r = b1.reshape(1, -1)
    b2r = b2.reshape(1, -1)
    b3p = jnp.pad(b3, (0, 16 - b3.shape[0])).reshape(1, -1)

    s1 = _mm_first(x, W1)
    p1 = _agg64(s1, ed)
    s2 = _mm_fused(p1, b1r, W2)
    p2 = _agg32(s2, ed)
    s3 = _mm_fused(p2, b2r, W3p)
    p3 = _agg16(s3, ed)
    outp = _final_add(p3, b3p)
    return outp[:, :W3.shape[1]]


# speedup vs baseline: 11.9907x; 1.4496x over previous
import functools

import jax
import jax.numpy as jnp
from jax import lax
from jax.experimental import pallas as pl
from jax.experimental.pallas import tpu as pltpu
from jax.experimental.pallas import tpu_sc as plsc

N = 10000
E = 320000
NCORES = 2
NSUB = 16
NTILES = NCORES * NSUB
EDGES_PER_TILE = E // NTILES
CHUNK = 80
NCHUNKS = EDGES_PER_TILE // CHUNK
SLAB = 624
TAIL = N - NSUB * SLAB
ZROWS = 208


NTRIPLES = 41


def _make_agg(D):
    mesh = plsc.VectorSubcoreMesh(core_axis_name="c", subcore_axis_name="s")

    @functools.partial(
        pl.kernel,
        out_type=jax.ShapeDtypeStruct((NCORES, N, D), jnp.float32),
        mesh=mesh,
        compiler_params=pltpu.CompilerParams(use_tc_tiling_on_sc=False,
                                             needs_layout_passes=False),
        scratch_types=[
            pltpu.VMEM((3, 3, CHUNK), jnp.int32),
            pltpu.VMEM((3, CHUNK), jnp.int32),
            pltpu.VMEM((3, CHUNK, D), jnp.float32),
            pltpu.VMEM((3, CHUNK, D), jnp.float32),
            pltpu.VMEM((ZROWS, D), jnp.float32),
            pltpu.VMEM_SHARED((N, D), jnp.float32),
            pltpu.SemaphoreType.DMA,
            pltpu.SemaphoreType.DMA,
            pltpu.SemaphoreType.DMA,
            pltpu.SemaphoreType.DMA,
            pltpu.SemaphoreType.DMA,
            pltpu.SemaphoreType.DMA,
            pltpu.SemaphoreType.DMA,
            pltpu.SemaphoreType.DMA,
            pltpu.SemaphoreType.DMA,
        ],
    )
    def agg(sup_hbm, ed_hbm, out_hbm,
            ebuf, dbuf, rows, srows, zbuf, acc,
            isem0, isem1, isem2, gsem0, gsem1, gsem2, ssem0, ssem1, ssem2):
        c = lax.axis_index("c")
        s = lax.axis_index("s")
        isem = (isem0, isem1, isem2)
        gsem = (gsem0, gsem1, gsem2)
        ssem = (ssem0, ssem1, ssem2)

        zv = jnp.zeros((16,), jnp.float32)

        def zrow(r, carry):
            for cb in range(D // 16):
                zbuf[r, pl.ds(cb * 16, 16)] = zv
            return carry

        lax.fori_loop(0, ZROWS, zrow, 0)

        row0 = s * SLAB
        for z in range(SLAB // ZROWS):
            pltpu.sync_copy(zbuf, acc.at[pl.ds(row0 + z * ZROWS, ZROWS)])

        @pl.when(s == NSUB - 1)
        def _zero_tail():
            pltpu.sync_copy(zbuf.at[pl.ds(0, TAIL)],
                            acc.at[pl.ds(NSUB * SLAB, TAIL)])

        plsc.subcore_barrier()

        base = (c * NSUB + s) * EDGES_PER_TILE

        def idx_cp(j, b):
            off = base + j * CHUNK
            return pltpu.make_async_copy(
                ed_hbm.at[:, pl.ds(off, CHUNK)], ebuf.at[b], isem[b])

        def gat_cp(b):
            return pltpu.make_async_copy(
                sup_hbm.at[ebuf.at[b, 0]], rows.at[b], gsem[b])

        def sct_start(b):
            pltpu.async_copy(srows.at[b], acc.at[dbuf.at[b]], ssem[b],
                             add=True)

        def sct_wait(b):
            pltpu.make_async_copy(srows.at[b], acc.at[dbuf.at[b]],
                                  ssem[b]).wait()

        blks = D // 16
        subrows = 16 // blks

        def process(b):
            @plsc.parallel_loop(0, CHUNK // 16, unroll=2)
            def _grp(g):
                dbuf[b, pl.ds(g * 16, 16)] = ebuf[b, 1, pl.ds(g * 16, 16)]
                wv = plsc.bitcast(ebuf[b, 2, pl.ds(g * 16, 16)], jnp.float32)
                for sub in range(16 // subrows):
                    vals = []
                    for i in range(subrows):
                        r = g * 16 + sub * subrows + i
                        for cb in range(blks):
                            vals.append(rows[b, r, pl.ds(cb * 16, 16)])
                    k = 0
                    for i in range(subrows):
                        w = wv[sub * subrows + i]
                        r = g * 16 + sub * subrows + i
                        for cb in range(blks):
                            srows[b, r, pl.ds(cb * 16, 16)] = vals[k] * w
                            k += 1

        idx_cp(0, 0).start()
        idx_cp(0, 0).wait()
        gat_cp(0).start()
        idx_cp(1, 1).start()
        idx_cp(2, 2).start()

        def triple(t, carry):
            for u in range(3):
                j = 3 * t + u
                nu = (u + 1) % 3
                gat_cp(u).wait()
                idx_cp(j + 1, nu).wait()
                gat_cp(nu).start()

                @pl.when(t > 0)
                def _():
                    sct_wait(u)

                process(u)
                sct_start(u)

                if u < 2:
                    idx_cp(j + 3, u).start()
                else:
                    @pl.when(t < NTRIPLES - 1)
                    def _():
                        idx_cp(j + 3, u).start()
            return carry

        lax.fori_loop(0, NTRIPLES, triple, 0)

        gat_cp(0).wait()
        idx_cp(124, 1).wait()
        gat_cp(1).start()
        sct_wait(0)
        process(0)
        sct_start(0)

        gat_cp(1).wait()
        sct_wait(1)
        process(1)
        sct_start(1)

        sct_wait(2)
        sct_wait(0)
        sct_wait(1)

        plsc.subcore_barrier()
        pltpu.sync_copy(acc.at[pl.ds(row0, SLAB)],
                        out_hbm.at[c, pl.ds(row0, SLAB)])

        @pl.when(s == NSUB - 1)
        def _drain_tail():
            pltpu.sync_copy(acc.at[pl.ds(NSUB * SLAB, TAIL)],
                            out_hbm.at[c, pl.ds(NSUB * SLAB, TAIL)])

    return agg


_agg64 = _make_agg(64)
_agg32 = _make_agg(32)
_agg16 = _make_agg(16)

_ROWS_BLK = 1000
_GRID = N // _ROWS_BLK


def _mm_first(x, W):
    K, Dout = W.shape

    def body(x_ref, w_ref, o_ref):
        o_ref[...] = lax.dot_general(
            x_ref[...], w_ref[...], (((1,), (0,)), ((), ())),
            precision=lax.Precision.HIGHEST)

    return pl.pallas_call(
        body,
        grid=(_GRID,),
        in_specs=[pl.BlockSpec((_ROWS_BLK, K), lambda i: (i, 0)),
                  pl.BlockSpec((K, Dout), lambda i: (0, 0))],
        out_specs=pl.BlockSpec((_ROWS_BLK, Dout), lambda i: (i, 0)),
        out_shape=jax.ShapeDtypeStruct((N, Dout), jnp.float32),
    )(x, W)


def _mm_fused(parts, b, W):
    Din, Dout = W.shape

    def body(p_ref, b_ref, w_ref, o_ref):
        h = jnp.maximum(p_ref[0] + p_ref[1] + b_ref[...], 0.0)
        o_ref[...] = lax.dot_general(
            h, w_ref[...], (((1,), (0,)), ((), ())),
            precision=lax.Precision.HIGHEST)

    return pl.pallas_call(
        body,
        grid=(_GRID,),
        in_specs=[pl.BlockSpec((NCORES, _ROWS_BLK, Din), lambda i: (0, i, 0)),
                  pl.BlockSpec((1, Din), lambda i: (0, 0)),
                  pl.BlockSpec((Din, Dout), lambda i: (0, 0))],
        out_specs=pl.BlockSpec((_ROWS_BLK, Dout), lambda i: (i, 0)),
        out_shape=jax.ShapeDtypeStruct((N, Dout), jnp.float32),
    )(parts, b, W)


def _final_add(parts, b):
    Din = parts.shape[-1]

    def body(p_ref, b_ref, o_ref):
        o_ref[...] = p_ref[0] + p_ref[1] + b_ref[...]

    return pl.pallas_call(
        body,
        grid=(_GRID,),
        in_specs=[pl.BlockSpec((NCORES, _ROWS_BLK, Din), lambda i: (0, i, 0)),
                  pl.BlockSpec((1, Din), lambda i: (0, 0))],
        out_specs=pl.BlockSpec((_ROWS_BLK, Din), lambda i: (i, 0)),
        out_shape=jax.ShapeDtypeStruct((N, Din), jnp.float32),
    )(parts, b)


def kernel(x, edge_index, edge_weight, W1, b1, W2, b2, W3, b3):
    wbits = jax.lax.bitcast_convert_type(edge_weight, jnp.int32)
    ed = jnp.concatenate([edge_index, wbits[None]], axis=0)
    W3p = jnp.pad(W3, ((0, 0), (0, 16 - W3.shape[1])))
    b1r = b1.reshape(1, -1)
    b2r = b2.reshape(1, -1)
    b3p = jnp.pad(b3, (0, 16 - b3.shape[0])).reshape(1, -1)

    s1 = _mm_first(x, W1)
    p1 = _agg64(s1, ed)
    s2 = _mm_fused(p1, b1r, W2)
    p2 = _agg32(s2, ed)
    s3 = _mm_fused(p2, b2r, W3p)
    p3 = _agg16(s3, ed)
    outp = _final_add(p3, b3p)
    return outp[:, :W3.shape[1]]


# speedup vs baseline: 13.2777x; 1.1073x over previous
import functools

import jax
import jax.numpy as jnp
from jax import lax
from jax.experimental import pallas as pl
from jax.experimental.pallas import tpu as pltpu
from jax.experimental.pallas import tpu_sc as plsc

N = 10000
E = 320000
NCORES = 2
NSUB = 16
NTILES = NCORES * NSUB
EDGES_PER_TILE = E // NTILES
CHUNK = 80
NCHUNKS = EDGES_PER_TILE // CHUNK
SLAB = 624
TAIL = N - NSUB * SLAB
ZROWS = 208


NTRIPLES = 41


def _make_agg(D):
    mesh = plsc.VectorSubcoreMesh(core_axis_name="c", subcore_axis_name="s")

    @functools.partial(
        pl.kernel,
        out_type=jax.ShapeDtypeStruct((NCORES, N, D), jnp.float32),
        mesh=mesh,
        compiler_params=pltpu.CompilerParams(use_tc_tiling_on_sc=False,
                                             needs_layout_passes=False),
        scratch_types=[
            pltpu.VMEM((3, 3, CHUNK), jnp.int32),
            pltpu.VMEM((3, CHUNK), jnp.int32),
            pltpu.VMEM((3, CHUNK, D), jnp.float32),
            pltpu.VMEM((3, CHUNK, D), jnp.float32),
            pltpu.VMEM((ZROWS, D), jnp.float32),
            pltpu.VMEM_SHARED((N, D), jnp.float32),
            pltpu.SemaphoreType.DMA,
            pltpu.SemaphoreType.DMA,
            pltpu.SemaphoreType.DMA,
            pltpu.SemaphoreType.DMA,
            pltpu.SemaphoreType.DMA,
            pltpu.SemaphoreType.DMA,
            pltpu.SemaphoreType.DMA,
            pltpu.SemaphoreType.DMA,
            pltpu.SemaphoreType.DMA,
        ],
    )
    def agg(sup_hbm, ed_hbm, out_hbm,
            ebuf, dbuf, rows, srows, zbuf, acc,
            isem0, isem1, isem2, gsem0, gsem1, gsem2, ssem0, ssem1, ssem2):
        c = lax.axis_index("c")
        s = lax.axis_index("s")
        isem = (isem0, isem1, isem2)
        gsem = (gsem0, gsem1, gsem2)
        ssem = (ssem0, ssem1, ssem2)

        zv = jnp.zeros((16,), jnp.float32)

        def zrow(r, carry):
            for cb in range(D // 16):
                zbuf[r, pl.ds(cb * 16, 16)] = zv
            return carry

        lax.fori_loop(0, ZROWS, zrow, 0)

        row0 = s * SLAB
        for z in range(SLAB // ZROWS):
            pltpu.sync_copy(zbuf, acc.at[pl.ds(row0 + z * ZROWS, ZROWS)])

        @pl.when(s == NSUB - 1)
        def _zero_tail():
            pltpu.sync_copy(zbuf.at[pl.ds(0, TAIL)],
                            acc.at[pl.ds(NSUB * SLAB, TAIL)])

        plsc.subcore_barrier()

        base = (c * NSUB + s) * EDGES_PER_TILE

        def idx_cp(j, b):
            off = base + j * CHUNK
            return pltpu.make_async_copy(
                ed_hbm.at[:, pl.ds(off, CHUNK)], ebuf.at[b], isem[b])

        def gat_cp(b):
            return pltpu.make_async_copy(
                sup_hbm.at[ebuf.at[b, 0]], rows.at[b], gsem[b])

        def sct_start(b):
            pltpu.async_copy(srows.at[b], acc.at[dbuf.at[b]], ssem[b],
                             add=True)

        def sct_wait(b):
            pltpu.make_async_copy(srows.at[b], acc.at[dbuf.at[b]],
                                  ssem[b]).wait()

        blks = D // 16
        subrows = 16 // blks

        def process(b):
            @plsc.parallel_loop(0, CHUNK // 16, unroll=2)
            def _grp(g):
                dbuf[b, pl.ds(g * 16, 16)] = ebuf[b, 1, pl.ds(g * 16, 16)]
                wv = plsc.bitcast(ebuf[b, 2, pl.ds(g * 16, 16)], jnp.float32)
                for sub in range(16 // subrows):
                    vals = []
                    for i in range(subrows):
                        r = g * 16 + sub * subrows + i
                        for cb in range(blks):
                            vals.append(rows[b, r, pl.ds(cb * 16, 16)])
                    k = 0
                    for i in range(subrows):
                        w = wv[sub * subrows + i]
                        r = g * 16 + sub * subrows + i
                        for cb in range(blks):
                            srows[b, r, pl.ds(cb * 16, 16)] = vals[k] * w
                            k += 1

        idx_cp(0, 0).start()
        idx_cp(1, 1).start()
        idx_cp(2, 2).start()
        idx_cp(0, 0).wait()
        gat_cp(0).start()
        idx_cp(1, 1).wait()
        gat_cp(1).start()

        def triple(t, carry):
            for u in range(3):
                j = 3 * t + u
                w = (u + 2) % 3
                gat_cp(u).wait()
                idx_cp(j + 2, w).wait()
                gat_cp(w).start()

                @pl.when(t > 0)
                def _():
                    sct_wait(u)

                process(u)
                sct_start(u)

                if u < 2:
                    idx_cp(j + 3, u).start()
                else:
                    @pl.when(t < NTRIPLES - 1)
                    def _():
                        idx_cp(j + 3, u).start()
            return carry

        lax.fori_loop(0, NTRIPLES, triple, 0)

        gat_cp(0).wait()
        sct_wait(0)
        process(0)
        sct_start(0)

        gat_cp(1).wait()
        sct_wait(1)
        process(1)
        sct_start(1)

        sct_wait(2)
        sct_wait(0)
        sct_wait(1)

        plsc.subcore_barrier()
        pltpu.sync_copy(acc.at[pl.ds(row0, SLAB)],
                        out_hbm.at[c, pl.ds(row0, SLAB)])

        @pl.when(s == NSUB - 1)
        def _drain_tail():
            pltpu.sync_copy(acc.at[pl.ds(NSUB * SLAB, TAIL)],
                            out_hbm.at[c, pl.ds(NSUB * SLAB, TAIL)])

    return agg


_agg64 = _make_agg(64)
_agg32 = _make_agg(32)
_agg16 = _make_agg(16)

_ROWS_BLK = 1000
_GRID = N // _ROWS_BLK


def _mm_first(x, W):
    K, Dout = W.shape

    def body(x_ref, w_ref, o_ref):
        o_ref[...] = lax.dot_general(
            x_ref[...], w_ref[...], (((1,), (0,)), ((), ())),
            precision=lax.Precision.HIGHEST)

    return pl.pallas_call(
        body,
        grid=(_GRID,),
        in_specs=[pl.BlockSpec((_ROWS_BLK, K), lambda i: (i, 0)),
                  pl.BlockSpec((K, Dout), lambda i: (0, 0))],
        out_specs=pl.BlockSpec((_ROWS_BLK, Dout), lambda i: (i, 0)),
        out_shape=jax.ShapeDtypeStruct((N, Dout), jnp.float32),
    )(x, W)


def _mm_fused(parts, b, W):
    Din, Dout = W.shape

    def body(p_ref, b_ref, w_ref, o_ref):
        h = jnp.maximum(p_ref[0] + p_ref[1] + b_ref[...], 0.0)
        o_ref[...] = lax.dot_general(
            h, w_ref[...], (((1,), (0,)), ((), ())),
            precision=lax.Precision.HIGHEST)

    return pl.pallas_call(
        body,
        grid=(_GRID,),
        in_specs=[pl.BlockSpec((NCORES, _ROWS_BLK, Din), lambda i: (0, i, 0)),
                  pl.BlockSpec((1, Din), lambda i: (0, 0)),
                  pl.BlockSpec((Din, Dout), lambda i: (0, 0))],
        out_specs=pl.BlockSpec((_ROWS_BLK, Dout), lambda i: (i, 0)),
        out_shape=jax.ShapeDtypeStruct((N, Dout), jnp.float32),
    )(parts, b, W)


def _final_add(parts, b):
    Din = parts.shape[-1]

    def body(p_ref, b_ref, o_ref):
        o_ref[...] = p_ref[0] + p_ref[1] + b_ref[...]

    return pl.pallas_call(
        body,
        grid=(_GRID,),
        in_specs=[pl.BlockSpec((NCORES, _ROWS_BLK, Din), lambda i: (0, i, 0)),
                  pl.BlockSpec((1, Din), lambda i: (0, 0))],
        out_specs=pl.BlockSpec((_ROWS_BLK, Din), lambda i: (i, 0)),
        out_shape=jax.ShapeDtypeStruct((N, Din), jnp.float32),
    )(parts, b)


def kernel(x, edge_index, edge_weight, W1, b1, W2, b2, W3, b3):
    wbits = jax.lax.bitcast_convert_type(edge_weight, jnp.int32)
    ed = jnp.concatenate([edge_index, wbits[None]], axis=0)
    W3p = jnp.pad(W3, ((0, 0), (0, 16 - W3.shape[1])))
    b1r = b1.reshape(1, -1)
    b2r = b2.reshape(1, -1)
    b3p = jnp.pad(b3, (0, 16 - b3.shape[0])).reshape(1, -1)

    s1 = _mm_first(x, W1)
    p1 = _agg64(s1, ed)
    s2 = _mm_fused(p1, b1r, W2)
    p2 = _agg32(s2, ed)
    s3 = _mm_fused(p2, b2r, W3p)
    p3 = _agg16(s3, ed)
    outp = _final_add(p3, b3p)
    return outp[:, :W3.shape[1]]


# speedup vs baseline: 13.8289x; 1.0415x over previous
import functools

import jax
import jax.numpy as jnp
from jax import lax
from jax.experimental import pallas as pl
from jax.experimental.pallas import tpu as pltpu
from jax.experimental.pallas import tpu_sc as plsc

N = 10000
E = 320000
NCORES = 2
NSUB = 16
NTILES = NCORES * NSUB
EDGES_PER_TILE = E // NTILES
CHUNK = 80
NCHUNKS = EDGES_PER_TILE // CHUNK
SLAB = 624
TAIL = N - NSUB * SLAB
ZROWS = 208


NTRIPLES = 41


def _make_agg(D):
    mesh = plsc.VectorSubcoreMesh(core_axis_name="c", subcore_axis_name="s")

    @functools.partial(
        pl.kernel,
        out_type=jax.ShapeDtypeStruct((NCORES, N, D), jnp.float32),
        mesh=mesh,
        compiler_params=pltpu.CompilerParams(use_tc_tiling_on_sc=False,
                                             needs_layout_passes=False,
                                             skip_device_barrier=True),
        scratch_types=[
            pltpu.VMEM((3, 3, CHUNK), jnp.int32),
            pltpu.VMEM((3, CHUNK), jnp.int32),
            pltpu.VMEM((3, CHUNK, D), jnp.float32),
            pltpu.VMEM((3, CHUNK, D), jnp.float32),
            pltpu.VMEM((ZROWS, D), jnp.float32),
            pltpu.VMEM_SHARED((N, D), jnp.float32),
            pltpu.SemaphoreType.DMA,
            pltpu.SemaphoreType.DMA,
            pltpu.SemaphoreType.DMA,
            pltpu.SemaphoreType.DMA,
            pltpu.SemaphoreType.DMA,
            pltpu.SemaphoreType.DMA,
            pltpu.SemaphoreType.DMA,
            pltpu.SemaphoreType.DMA,
            pltpu.SemaphoreType.DMA,
        ],
    )
    def agg(sup_hbm, ed_hbm, out_hbm,
            ebuf, dbuf, rows, srows, zbuf, acc,
            isem0, isem1, isem2, gsem0, gsem1, gsem2, ssem0, ssem1, ssem2):
        c = lax.axis_index("c")
        s = lax.axis_index("s")
        isem = (isem0, isem1, isem2)
        gsem = (gsem0, gsem1, gsem2)
        ssem = (ssem0, ssem1, ssem2)

        zv = jnp.zeros((16,), jnp.float32)

        def zrow(r, carry):
            for cb in range(D // 16):
                zbuf[r, pl.ds(cb * 16, 16)] = zv
            return carry

        lax.fori_loop(0, ZROWS, zrow, 0)

        row0 = s * SLAB
        for z in range(SLAB // ZROWS):
            pltpu.sync_copy(zbuf, acc.at[pl.ds(row0 + z * ZROWS, ZROWS)])

        @pl.when(s == NSUB - 1)
        def _zero_tail():
            pltpu.sync_copy(zbuf.at[pl.ds(0, TAIL)],
                            acc.at[pl.ds(NSUB * SLAB, TAIL)])

        plsc.subcore_barrier()

        base = (c * NSUB + s) * EDGES_PER_TILE

        def idx_cp(j, b):
            off = base + j * CHUNK
            return pltpu.make_async_copy(
                ed_hbm.at[:, pl.ds(off, CHUNK)], ebuf.at[b], isem[b])

        def gat_cp(b):
            return pltpu.make_async_copy(
                sup_hbm.at[ebuf.at[b, 0]], rows.at[b], gsem[b])

        def sct_start(b):
            pltpu.async_copy(srows.at[b], acc.at[dbuf.at[b]], ssem[b],
                             add=True)

        def sct_wait(b):
            pltpu.make_async_copy(srows.at[b], acc.at[dbuf.at[b]],
                                  ssem[b]).wait()

        blks = D // 16
        subrows = 16 // blks

        def process(b):
            @plsc.parallel_loop(0, CHUNK // 16, unroll=2)
            def _grp(g):
                dbuf[b, pl.ds(g * 16, 16)] = ebuf[b, 1, pl.ds(g * 16, 16)]
                wv = plsc.bitcast(ebuf[b, 2, pl.ds(g * 16, 16)], jnp.float32)
                for sub in range(16 // subrows):
                    vals = []
                    for i in range(subrows):
                        r = g * 16 + sub * subrows + i
                        for cb in range(blks):
                            vals.append(rows[b, r, pl.ds(cb * 16, 16)])
                    k = 0
                    for i in range(subrows):
                        w = wv[sub * subrows + i]
                        r = g * 16 + sub * subrows + i
                        for cb in range(blks):
                            srows[b, r, pl.ds(cb * 16, 16)] = vals[k] * w
                            k += 1

        idx_cp(0, 0).start()
        idx_cp(1, 1).start()
        idx_cp(2, 2).start()
        idx_cp(0, 0).wait()
        gat_cp(0).start()
        idx_cp(1, 1).wait()
        gat_cp(1).start()

        def triple(t, carry):
            for u in range(3):
                j = 3 * t + u
                w = (u + 2) % 3
                gat_cp(u).wait()
                idx_cp(j + 2, w).wait()
                gat_cp(w).start()

                @pl.when(t > 0)
                def _():
                    sct_wait(u)

                process(u)
                sct_start(u)

                if u < 2:
                    idx_cp(j + 3, u).start()
                else:
                    @pl.when(t < NTRIPLES - 1)
                    def _():
                        idx_cp(j + 3, u).start()
            return carry

        lax.fori_loop(0, NTRIPLES, triple, 0)

        gat_cp(0).wait()
        sct_wait(0)
        process(0)
        sct_start(0)

        gat_cp(1).wait()
        sct_wait(1)
        process(1)
        sct_start(1)

        sct_wait(2)
        sct_wait(0)
        sct_wait(1)

        plsc.subcore_barrier()
        pltpu.sync_copy(acc.at[pl.ds(row0, SLAB)],
                        out_hbm.at[c, pl.ds(row0, SLAB)])

        @pl.when(s == NSUB - 1)
        def _drain_tail():
            pltpu.sync_copy(acc.at[pl.ds(NSUB * SLAB, TAIL)],
                            out_hbm.at[c, pl.ds(NSUB * SLAB, TAIL)])

    return agg


_agg64 = _make_agg(64)
_agg32 = _make_agg(32)
_agg16 = _make_agg(16)

_ROWS_BLK = 2000
_GRID = N // _ROWS_BLK


def _mm_first(x, W):
    K, Dout = W.shape

    def body(x_ref, w_ref, o_ref):
        o_ref[...] = lax.dot_general(
            x_ref[...], w_ref[...], (((1,), (0,)), ((), ())),
            precision=lax.Precision.HIGHEST)

    return pl.pallas_call(
        body,
        grid=(_GRID,),
        in_specs=[pl.BlockSpec((_ROWS_BLK, K), lambda i: (i, 0)),
                  pl.BlockSpec((K, Dout), lambda i: (0, 0))],
        out_specs=pl.BlockSpec((_ROWS_BLK, Dout), lambda i: (i, 0)),
        out_shape=jax.ShapeDtypeStruct((N, Dout), jnp.float32),
    )(x, W)


def _mm_fused(parts, b, W):
    Din, Dout = W.shape

    def body(p_ref, b_ref, w_ref, o_ref):
        h = jnp.maximum(p_ref[0] + p_ref[1] + b_ref[...], 0.0)
        o_ref[...] = lax.dot_general(
            h, w_ref[...], (((1,), (0,)), ((), ())),
            precision=lax.Precision.HIGHEST)

    return pl.pallas_call(
        body,
        grid=(_GRID,),
        in_specs=[pl.BlockSpec((NCORES, _ROWS_BLK, Din), lambda i: (0, i, 0)),
                  pl.BlockSpec((1, Din), lambda i: (0, 0)),
                  pl.BlockSpec((Din, Dout), lambda i: (0, 0))],
        out_specs=pl.BlockSpec((_ROWS_BLK, Dout), lambda i: (i, 0)),
        out_shape=jax.ShapeDtypeStruct((N, Dout), jnp.float32),
    )(parts, b, W)


def _final_add(parts, b):
    Din = parts.shape[-1]

    def body(p_ref, b_ref, o_ref):
        o_ref[...] = p_ref[0] + p_ref[1] + b_ref[...]

    return pl.pallas_call(
        body,
        grid=(_GRID,),
        in_specs=[pl.BlockSpec((NCORES, _ROWS_BLK, Din), lambda i: (0, i, 0)),
                  pl.BlockSpec((1, Din), lambda i: (0, 0))],
        out_specs=pl.BlockSpec((_ROWS_BLK, Din), lambda i: (i, 0)),
        out_shape=jax.ShapeDtypeStruct((N, Din), jnp.float32),
    )(parts, b)


def kernel(x, edge_index, edge_weight, W1, b1, W2, b2, W3, b3):
    wbits = jax.lax.bitcast_convert_type(edge_weight, jnp.int32)
    ed = jnp.concatenate([edge_index, wbits[None]], axis=0)
    W3p = jnp.pad(W3, ((0, 0), (0, 16 - W3.shape[1])))
    b1r = b1.reshape(1, -1)
    b2r = b2.reshape(1, -1)
    b3p = jnp.pad(b3, (0, 16 - b3.shape[0])).reshape(1, -1)

    s1 = _mm_first(x, W1)
    p1 = _agg64(s1, ed)
    s2 = _mm_fused(p1, b1r, W2)
    p2 = _agg32(s2, ed)
    s3 = _mm_fused(p2, b2r, W3p)
    p3 = _agg16(s3, ed)
    outp = _final_add(p3, b3p)
    return outp[:, :W3.shape[1]]


# speedup vs baseline: 13.9233x; 1.0068x over previous
import functools

import jax
import jax.numpy as jnp
from jax import lax
from jax.experimental import pallas as pl
from jax.experimental.pallas import tpu as pltpu
from jax.experimental.pallas import tpu_sc as plsc

N = 10000
E = 320000
NCORES = 2
NSUB = 16
NTILES = NCORES * NSUB
EDGES_PER_TILE = E // NTILES
CHUNK = 80
NCHUNKS = EDGES_PER_TILE // CHUNK
SLAB = 624
TAIL = N - NSUB * SLAB
ZROWS = 208


NTRIPLES = 41


def _make_agg(D):
    mesh = plsc.VectorSubcoreMesh(core_axis_name="c", subcore_axis_name="s")

    @functools.partial(
        pl.kernel,
        out_type=jax.ShapeDtypeStruct((NCORES, N, D), jnp.float32),
        mesh=mesh,
        compiler_params=pltpu.CompilerParams(use_tc_tiling_on_sc=False,
                                             needs_layout_passes=False,
                                             skip_device_barrier=True),
        scratch_types=[
            pltpu.VMEM((3, 3, CHUNK), jnp.int32),
            pltpu.VMEM((3, CHUNK), jnp.int32),
            pltpu.VMEM((3, CHUNK, D), jnp.float32),
            pltpu.VMEM((3, CHUNK, D), jnp.float32),
            pltpu.VMEM((ZROWS, D), jnp.float32),
            pltpu.VMEM_SHARED((N, D), jnp.float32),
            pltpu.SemaphoreType.DMA,
            pltpu.SemaphoreType.DMA,
            pltpu.SemaphoreType.DMA,
            pltpu.SemaphoreType.DMA,
            pltpu.SemaphoreType.DMA,
            pltpu.SemaphoreType.DMA,
            pltpu.SemaphoreType.DMA,
            pltpu.SemaphoreType.DMA,
            pltpu.SemaphoreType.DMA,
        ],
    )
    def agg(sup_hbm, ed_hbm, out_hbm,
            ebuf, dbuf, rows, srows, zbuf, acc,
            isem0, isem1, isem2, gsem0, gsem1, gsem2, ssem0, ssem1, ssem2):
        c = lax.axis_index("c")
        s = lax.axis_index("s")
        isem = (isem0, isem1, isem2)
        gsem = (gsem0, gsem1, gsem2)
        ssem = (ssem0, ssem1, ssem2)

        zv = jnp.zeros((16,), jnp.float32)

        def zrow(r, carry):
            for cb in range(D // 16):
                zbuf[r, pl.ds(cb * 16, 16)] = zv
            return carry

        lax.fori_loop(0, ZROWS, zrow, 0)

        row0 = s * SLAB
        for z in range(SLAB // ZROWS):
            pltpu.sync_copy(zbuf, acc.at[pl.ds(row0 + z * ZROWS, ZROWS)])

        @pl.when(s == NSUB - 1)
        def _zero_tail():
            pltpu.sync_copy(zbuf.at[pl.ds(0, TAIL)],
                            acc.at[pl.ds(NSUB * SLAB, TAIL)])

        plsc.subcore_barrier()

        base = (c * NSUB + s) * EDGES_PER_TILE

        def idx_cp(j, b):
            off = base + j * CHUNK
            return pltpu.make_async_copy(
                ed_hbm.at[:, pl.ds(off, CHUNK)], ebuf.at[b], isem[b])

        def gat_cp(b):
            return pltpu.make_async_copy(
                sup_hbm.at[ebuf.at[b, 0]], rows.at[b], gsem[b])

        def sct_start(b):
            pltpu.async_copy(srows.at[b], acc.at[dbuf.at[b]], ssem[b],
                             add=True)

        def sct_wait(b):
            pltpu.make_async_copy(srows.at[b], acc.at[dbuf.at[b]],
                                  ssem[b]).wait()

        blks = D // 16
        subrows = 16 // blks

        def process(b):
            @plsc.parallel_loop(0, CHUNK // 16, unroll=4)
            def _grp(g):
                dbuf[b, pl.ds(g * 16, 16)] = ebuf[b, 1, pl.ds(g * 16, 16)]
                wv = plsc.bitcast(ebuf[b, 2, pl.ds(g * 16, 16)], jnp.float32)
                for sub in range(16 // subrows):
                    vals = []
                    for i in range(subrows):
                        r = g * 16 + sub * subrows + i
                        for cb in range(blks):
                            vals.append(rows[b, r, pl.ds(cb * 16, 16)])
                    k = 0
                    for i in range(subrows):
                        w = wv[sub * subrows + i]
                        r = g * 16 + sub * subrows + i
                        for cb in range(blks):
                            srows[b, r, pl.ds(cb * 16, 16)] = vals[k] * w
                            k += 1

        idx_cp(0, 0).start()
        idx_cp(1, 1).start()
        idx_cp(2, 2).start()
        idx_cp(0, 0).wait()
        gat_cp(0).start()
        idx_cp(1, 1).wait()
        gat_cp(1).start()

        def triple(t, carry):
            for u in range(3):
                j = 3 * t + u
                w = (u + 2) % 3
                gat_cp(u).wait()
                idx_cp(j + 2, w).wait()
                gat_cp(w).start()

                @pl.when(t > 0)
                def _():
                    sct_wait(u)

                process(u)
                sct_start(u)

                if u < 2:
                    idx_cp(j + 3, u).start()
                else:
                    @pl.when(t < NTRIPLES - 1)
                    def _():
                        idx_cp(j + 3, u).start()
            return carry

        lax.fori_loop(0, NTRIPLES, triple, 0)

        gat_cp(0).wait()
        sct_wait(0)
        process(0)
        sct_start(0)

        gat_cp(1).wait()
        sct_wait(1)
        process(1)
        sct_start(1)

        sct_wait(2)
        sct_wait(0)
        sct_wait(1)

        plsc.subcore_barrier()
        pltpu.sync_copy(acc.at[pl.ds(row0, SLAB)],
                        out_hbm.at[c, pl.ds(row0, SLAB)])

        @pl.when(s == NSUB - 1)
        def _drain_tail():
            pltpu.sync_copy(acc.at[pl.ds(NSUB * SLAB, TAIL)],
                            out_hbm.at[c, pl.ds(NSUB * SLAB, TAIL)])

    return agg


_agg64 = _make_agg(64)
_agg32 = _make_agg(32)
_agg16 = _make_agg(16)

_ROWS_BLK = 2000
_GRID = N // _ROWS_BLK


def _mm_first(x, W):
    K, Dout = W.shape

    def body(x_ref, w_ref, o_ref):
        o_ref[...] = lax.dot_general(
            x_ref[...], w_ref[...], (((1,), (0,)), ((), ())),
            precision=lax.Precision.HIGHEST)

    return pl.pallas_call(
        body,
        grid=(_GRID,),
        in_specs=[pl.BlockSpec((_ROWS_BLK, K), lambda i: (i, 0)),
                  pl.BlockSpec((K, Dout), lambda i: (0, 0))],
        out_specs=pl.BlockSpec((_ROWS_BLK, Dout), lambda i: (i, 0)),
        out_shape=jax.ShapeDtypeStruct((N, Dout), jnp.float32),
    )(x, W)


def _mm_fused(parts, b, W):
    Din, Dout = W.shape

    def body(p_ref, b_ref, w_ref, o_ref):
        h = jnp.maximum(p_ref[0] + p_ref[1] + b_ref[...], 0.0)
        o_ref[...] = lax.dot_general(
            h, w_ref[...], (((1,), (0,)), ((), ())),
            precision=lax.Precision.HIGHEST)

    return pl.pallas_call(
        body,
        grid=(_GRID,),
        in_specs=[pl.BlockSpec((NCORES, _ROWS_BLK, Din), lambda i: (0, i, 0)),
                  pl.BlockSpec((1, Din), lambda i: (0, 0)),
                  pl.BlockSpec((Din, Dout), lambda i: (0, 0))],
        out_specs=pl.BlockSpec((_ROWS_BLK, Dout), lambda i: (i, 0)),
        out_shape=jax.ShapeDtypeStruct((N, Dout), jnp.float32),
    )(parts, b, W)


def _final_add(parts, b):
    Din = parts.shape[-1]

    def body(p_ref, b_ref, o_ref):
        o_ref[...] = p_ref[0] + p_ref[1] + b_ref[...]

    return pl.pallas_call(
        body,
        grid=(_GRID,),
        in_specs=[pl.BlockSpec((NCORES, _ROWS_BLK, Din), lambda i: (0, i, 0)),
                  pl.BlockSpec((1, Din), lambda i: (0, 0))],
        out_specs=pl.BlockSpec((_ROWS_BLK, Din), lambda i: (i, 0)),
        out_shape=jax.ShapeDtypeStruct((N, Din), jnp.float32),
    )(parts, b)


def kernel(x, edge_index, edge_weight, W1, b1, W2, b2, W3, b3):
    wbits = jax.lax.bitcast_convert_type(edge_weight, jnp.int32)
    ed = jnp.concatenate([edge_index, wbits[None]], axis=0)
    W3p = jnp.pad(W3, ((0, 0), (0, 16 - W3.shape[1])))
    b1r = b1.reshape(1, -1)
    b2r = b2.reshape(1, -1)
    b3p = jnp.pad(b3, (0, 16 - b3.shape[0])).reshape(1, -1)

    s1 = _mm_first(x, W1)
    p1 = _agg64(s1, ed)
    s2 = _mm_fused(p1, b1r, W2)
    p2 = _agg32(s2, ed)
    s3 = _mm_fused(p2, b2r, W3p)
    p3 = _agg16(s3, ed)
    outp = _final_add(p3, b3p)
    return outp[:, :W3.shape[1]]
